# Initial kernel scaffold; baseline (speedup 1.0000x reference)
import functools

import jax
import jax.numpy as jnp
from jax import lax
from jax.experimental import pallas as pl
from jax.experimental.pallas import tpu as pltpu
from jax.experimental.pallas import tpu_sc as plsc


_BR = 1000


def _linear_body(x_ref, w_ref, b_ref, o_ref):
    o_ref[...] = lax.dot_general(
        x_ref[...], w_ref[...], (((1,), (1,)), ((), ())),
        preferred_element_type=jnp.float32) + b_ref[...]


def _tc_linear(x, w, b):
    n, d = x.shape
    return pl.pallas_call(
        _linear_body,
        grid=(n // _BR,),
        in_specs=[
            pl.BlockSpec((_BR, d), lambda i: (i, 0)),
            pl.BlockSpec((d, d), lambda i: (0, 0)),
            pl.BlockSpec((1, d), lambda i: (0, 0)),
        ],
        out_specs=pl.BlockSpec((_BR, d), lambda i: (i, 0)),
        out_shape=jax.ShapeDtypeStruct((n, d), jnp.float32),
    )(x, w, b.reshape(1, d))


def _combine_body(p_ref, h_ref, wrel_ref, brel_ref, wroot_ref, o_ref):
    agg = p_ref[0] + p_ref[1]
    acc = lax.dot_general(agg, wrel_ref[...], (((1,), (1,)), ((), ())),
                          preferred_element_type=jnp.float32)
    acc += lax.dot_general(h_ref[...], wroot_ref[...], (((1,), (1,)), ((), ())),
                           preferred_element_type=jnp.float32)
    o_ref[...] = jnp.maximum(acc + brel_ref[...], 0.0)


def _tc_combine(p, h, w_rel, b_rel, w_root):
    n, d = h.shape
    return pl.pallas_call(
        _combine_body,
        grid=(n // _BR,),
        in_specs=[
            pl.BlockSpec((2, _BR, d), lambda i: (0, i, 0)),
            pl.BlockSpec((_BR, d), lambda i: (i, 0)),
            pl.BlockSpec((d, d), lambda i: (0, 0)),
            pl.BlockSpec((1, d), lambda i: (0, 0)),
            pl.BlockSpec((d, d), lambda i: (0, 0)),
        ],
        out_specs=pl.BlockSpec((_BR, d), lambda i: (i, 0)),
        out_shape=jax.ShapeDtypeStruct((n, d), jnp.float32),
    )(p, h, w_rel, b_rel.reshape(1, d), w_root)


_B = 80
_GRP = 5


def _sc_segsum_body(nc, ns, e_per_w, nb, n, d,
                    h_hbm, src_hbm, dst_hbm, out_hbm,
                    src_v, dst_v, rows_v, zb_v, acc_s, sem):
    c = lax.axis_index("c")
    s = lax.axis_index("s")
    wid = c * ns + s
    ebase = pl.multiple_of(wid * e_per_w, e_per_w)

    pltpu.sync_copy(src_hbm.at[pl.ds(ebase, e_per_w)], src_v)
    pltpu.sync_copy(dst_hbm.at[pl.ds(wid * nb, nb)], dst_v)

    zr = zb_v.shape[0]
    rows_per_tile = n // ns
    rbase = pl.multiple_of(s * rows_per_tile, rows_per_tile)
    for i in range(zr):
        for k in range(d // 16):
            zb_v[i, pl.ds(k * 16, 16)] = jnp.zeros((16,), jnp.float32)
    for k in range(rows_per_tile // zr):
        pltpu.sync_copy(zb_v, acc_s.at[pl.ds(rbase + k * zr, zr)])
    plsc.subcore_barrier()

    def body(jo, carry):
        obase = pl.multiple_of(jo * (_GRP * _B), _GRP * _B)
        descs = []
        for b in range(_GRP):
            descs.append(pltpu.async_copy(
                h_hbm.at[src_v.at[pl.ds(obase + b * _B, _B)]],
                rows_v.at[b], sem))
        for dsc in descs:
            dsc.wait()
        for b in range(_GRP):
            pltpu.sync_copy(rows_v.at[b], acc_s.at[dst_v.at[jo * _GRP + b]],
                            add=True)
        return carry

    lax.fori_loop(0, nb // _GRP, body, 0)
    plsc.subcore_barrier()

    wb = 125
    for k in range(rows_per_tile // wb):
        r0 = rbase + k * wb
        pltpu.sync_copy(acc_s.at[pl.ds(r0, wb)], out_hbm.at[c, pl.ds(r0, wb)])


def _sc_segment_sum(h, src, dst2):
    n, d = h.shape
    e = src.shape[0]
    info = plsc.get_sparse_core_info()
    nc, ns = info.num_cores, info.num_subcores
    nw = nc * ns
    assert e % (nw * _B) == 0 and (e // nw) % (_GRP * _B) == 0
    e_per_w = e // nw
    nb = e_per_w // _B
    mesh = plsc.VectorSubcoreMesh(core_axis_name="c", subcore_axis_name="s")
    kern = pl.kernel(
        functools.partial(_sc_segsum_body, nc, ns, e_per_w, nb, n, d),
        out_type=jax.ShapeDtypeStruct((nc, n, d), jnp.float32),
        mesh=mesh,
        scratch_types=[
            pltpu.VMEM((e_per_w,), jnp.int32),
            pltpu.VMEM((nb, _B), jnp.int32),
            pltpu.VMEM((_GRP, _B, d), jnp.float32),
            pltpu.VMEM((25, d), jnp.float32),
            pltpu.VMEM_SHARED((n, d), jnp.float32),
            pltpu.SemaphoreType.DMA,
        ],
    )
    return kern(h, src, dst2)


def kernel(x, edge_index, W_in, b_in, W_rel1, b_rel1, W_root1,
           W_rel2, b_rel2, W_root2):
    n, d = x.shape
    e = edge_index.shape[1]
    src = edge_index[0]
    dst2 = edge_index[1].reshape(e // _B, _B)

    h0 = _tc_linear(x, W_in, b_in)
    p1 = _sc_segment_sum(h0, src, dst2)
    h1 = _tc_combine(p1, h0, W_rel1, b_rel1, W_root1)
    p2 = _sc_segment_sum(h1, src, dst2)
    h2 = _tc_combine(p2, h1, W_rel2, b_rel2, W_root2)
    return h2


# speedup vs baseline: 6.7756x; 6.7756x over previous
import functools

import jax
import jax.numpy as jnp
from jax import lax
from jax.experimental import pallas as pl
from jax.experimental.pallas import tpu as pltpu
from jax.experimental.pallas import tpu_sc as plsc


_BR = 1000


def _linear_body(x_ref, w_ref, b_ref, o_ref):
    o_ref[...] = lax.dot_general(
        x_ref[...], w_ref[...], (((1,), (1,)), ((), ())),
        preferred_element_type=jnp.float32) + b_ref[...]


def _tc_linear(x, w, b):
    n, d = x.shape
    return pl.pallas_call(
        _linear_body,
        grid=(n // _BR,),
        in_specs=[
            pl.BlockSpec((_BR, d), lambda i: (i, 0)),
            pl.BlockSpec((d, d), lambda i: (0, 0)),
            pl.BlockSpec((1, d), lambda i: (0, 0)),
        ],
        out_specs=pl.BlockSpec((_BR, d), lambda i: (i, 0)),
        out_shape=jax.ShapeDtypeStruct((n, d), jnp.float32),
    )(x, w, b.reshape(1, d))


def _combine_body(nc, p_ref, h_ref, wrel_ref, brel_ref, wroot_ref, o_ref):
    agg = p_ref[0]
    for i in range(1, nc):
        agg = agg + p_ref[i]
    acc = lax.dot_general(agg, wrel_ref[...], (((1,), (1,)), ((), ())),
                          preferred_element_type=jnp.float32)
    acc += lax.dot_general(h_ref[...], wroot_ref[...], (((1,), (1,)), ((), ())),
                           preferred_element_type=jnp.float32)
    o_ref[...] = jnp.maximum(acc + brel_ref[...], 0.0)


def _tc_combine(p, h, w_rel, b_rel, w_root):
    nc, n, d = p.shape
    return pl.pallas_call(
        functools.partial(_combine_body, nc),
        grid=(n // _BR,),
        in_specs=[
            pl.BlockSpec((nc, _BR, d), lambda i: (0, i, 0)),
            pl.BlockSpec((_BR, d), lambda i: (i, 0)),
            pl.BlockSpec((d, d), lambda i: (0, 0)),
            pl.BlockSpec((1, d), lambda i: (0, 0)),
            pl.BlockSpec((d, d), lambda i: (0, 0)),
        ],
        out_specs=pl.BlockSpec((_BR, d), lambda i: (i, 0)),
        out_shape=jax.ShapeDtypeStruct((n, d), jnp.float32),
    )(p, h, w_rel, b_rel.reshape(1, d), w_root)


_B = 80
_CB = 25
_NCU = 1
_ZR = 16
_WB = 80


def _sc_segsum_body(ns, e_per_w, nchunks, n,
                    h_hbm, src_hbm, dst_hbm, out_hbm,
                    src0_v, src1_v, dst0_v, dst1_v, rows_v, zb_v, acc_s,
                    sg0, sg1, ss):
    src_b = (src0_v, src1_v)
    dst_b = (dst0_v, dst1_v)
    c = lax.axis_index("c")
    s = lax.axis_index("s")
    wid = c * ns + s
    ebase = pl.multiple_of(wid * e_per_w, e_per_w)
    cedges = _CB * _B
    sgs = (sg0, sg1)

    for i in range(_ZR):
        for k in range(zb_v.shape[1] // 16):
            zb_v[i, pl.ds(k * 16, 16)] = jnp.zeros((16,), jnp.float32)
    nwb = n // _WB
    for k in range((nwb + ns - 1) // ns):
        ci = k * ns + s

        @pl.when(ci < nwb)
        def _():
            r0 = ci * _WB
            for m in range(_WB // _ZR):
                pltpu.sync_copy(zb_v, acc_s.at[pl.ds(r0 + m * _ZR, _ZR)])
    plsc.subcore_barrier()

    def stage(cc, p, sync):
        off = pl.multiple_of(ebase + cc * cedges, cedges)
        if sync:
            pltpu.sync_copy(src_hbm.at[pl.ds(off, cedges)], src_b[p])
            pltpu.sync_copy(dst_hbm.at[wid, cc], dst_b[p])
        else:
            pltpu.async_copy(src_hbm.at[pl.ds(off, cedges)], src_b[p], ss)
            pltpu.async_copy(dst_hbm.at[wid, cc], dst_b[p], ss)

    def drain_stage(p):
        pltpu.make_async_copy(src_hbm.at[pl.ds(ebase, cedges)],
                              src_b[p], ss).wait()
        pltpu.make_async_copy(dst_hbm.at[wid, 0], dst_b[p], ss).wait()

    def process(p):
        def fire(j):
            return pltpu.async_copy(
                h_hbm.at[src_b[p].at[pl.ds(j * _B, _B)]],
                rows_v.at[j % 2], sgs[j % 2])

        descs = [None] * _CB
        descs[0] = fire(0)
        for j in range(_CB):
            if j + 1 < _CB:
                descs[j + 1] = fire(j + 1)
            descs[j].wait()
            pltpu.sync_copy(rows_v.at[j % 2], acc_s.at[dst_b[p].at[j]],
                            add=True)

    stage(0, 0, sync=True)

    def body(ci2, carry):
        stage(2 * ci2 + 1, 1, sync=False)
        process(0)
        drain_stage(1)

        @pl.when(ci2 < nchunks // 2 - 1)
        def _():
            stage(2 * ci2 + 2, 0, sync=False)
        process(1)

        @pl.when(ci2 < nchunks // 2 - 1)
        def _():
            drain_stage(0)
        return carry

    lax.fori_loop(0, nchunks // 2, body, 0)
    plsc.subcore_barrier()

    for k in range((nwb + ns - 1) // ns):
        ci = k * ns + s

        @pl.when(ci < nwb)
        def _():
            r0 = ci * _WB
            pltpu.sync_copy(acc_s.at[pl.ds(r0, _WB)],
                            out_hbm.at[c, pl.ds(r0, _WB)])


def _sc_segment_sum(h, src, dst4):
    n, d = h.shape
    e = src.shape[0]
    nw, nchunks, cb, b = dst4.shape
    info = plsc.get_sparse_core_info()
    ns = info.num_subcores
    assert nw == _NCU * ns and cb == _CB and b == _B
    assert nchunks % 2 == 0 and n % _WB == 0 and _WB % _ZR == 0
    e_per_w = e // nw
    mesh = plsc.VectorSubcoreMesh(core_axis_name="c", subcore_axis_name="s",
                                  num_cores=_NCU)
    kern = pl.kernel(
        functools.partial(_sc_segsum_body, ns, e_per_w, nchunks, n),
        out_type=jax.ShapeDtypeStruct((_NCU, n, d), jnp.float32),
        mesh=mesh,
        scratch_types=[
            pltpu.VMEM((_CB * _B,), jnp.int32),
            pltpu.VMEM((_CB * _B,), jnp.int32),
            pltpu.VMEM((_CB, _B), jnp.int32),
            pltpu.VMEM((_CB, _B), jnp.int32),
            pltpu.VMEM((2, _B, d), jnp.float32),
            pltpu.VMEM((_ZR, d), jnp.float32),
            pltpu.VMEM_SHARED((n, d), jnp.float32),
            pltpu.SemaphoreType.DMA,
            pltpu.SemaphoreType.DMA,
            pltpu.SemaphoreType.DMA,
        ],
    )
    return kern(h, src, dst4)


def kernel(x, edge_index, W_in, b_in, W_rel1, b_rel1, W_root1,
           W_rel2, b_rel2, W_root2):
    e = edge_index.shape[1]
    info = plsc.get_sparse_core_info()
    nw = _NCU * info.num_subcores
    assert e % (nw * _CB * _B) == 0
    src = edge_index[0]
    dst4 = edge_index[1].reshape(nw, e // (nw * _CB * _B), _CB, _B)

    h0 = _tc_linear(x, W_in, b_in)
    p1 = _sc_segment_sum(h0, src, dst4)
    h1 = _tc_combine(p1, h0, W_rel1, b_rel1, W_root1)
    p2 = _sc_segment_sum(h1, src, dst4)
    h2 = _tc_combine(p2, h1, W_rel2, b_rel2, W_root2)
    return h2


# speedup vs baseline: 6.7969x; 1.0031x over previous
import functools

import jax
import jax.numpy as jnp
from jax import lax
from jax.experimental import pallas as pl
from jax.experimental.pallas import tpu as pltpu
from jax.experimental.pallas import tpu_sc as plsc


_BR = 1000


def _linear_body(x_ref, w_ref, b_ref, o_ref):
    o_ref[...] = lax.dot_general(
        x_ref[...], w_ref[...], (((1,), (1,)), ((), ())),
        preferred_element_type=jnp.float32) + b_ref[...]


def _tc_linear(x, w, b):
    n, d = x.shape
    return pl.pallas_call(
        _linear_body,
        grid=(n // _BR,),
        in_specs=[
            pl.BlockSpec((_BR, d), lambda i: (i, 0)),
            pl.BlockSpec((d, d), lambda i: (0, 0)),
            pl.BlockSpec((1, d), lambda i: (0, 0)),
        ],
        out_specs=pl.BlockSpec((_BR, d), lambda i: (i, 0)),
        out_shape=jax.ShapeDtypeStruct((n, d), jnp.float32),
    )(x, w, b.reshape(1, d))


def _combine_body(nc, p_ref, h_ref, wrel_ref, brel_ref, wroot_ref, o_ref):
    agg = p_ref[0]
    for i in range(1, nc):
        agg = agg + p_ref[i]
    acc = lax.dot_general(agg, wrel_ref[...], (((1,), (1,)), ((), ())),
                          preferred_element_type=jnp.float32)
    acc += lax.dot_general(h_ref[...], wroot_ref[...], (((1,), (1,)), ((), ())),
                           preferred_element_type=jnp.float32)
    o_ref[...] = jnp.maximum(acc + brel_ref[...], 0.0)


def _tc_combine(p, h, w_rel, b_rel, w_root):
    nc, n, d = p.shape
    return pl.pallas_call(
        functools.partial(_combine_body, nc),
        grid=(n // _BR,),
        in_specs=[
            pl.BlockSpec((nc, _BR, d), lambda i: (0, i, 0)),
            pl.BlockSpec((_BR, d), lambda i: (i, 0)),
            pl.BlockSpec((d, d), lambda i: (0, 0)),
            pl.BlockSpec((1, d), lambda i: (0, 0)),
            pl.BlockSpec((d, d), lambda i: (0, 0)),
        ],
        out_specs=pl.BlockSpec((_BR, d), lambda i: (i, 0)),
        out_shape=jax.ShapeDtypeStruct((n, d), jnp.float32),
    )(p, h, w_rel, b_rel.reshape(1, d), w_root)


_B = 80
_CB = 25
_NCU = 1
_ZR = 16
_WB = 80


def _sc_segsum_body(ns, e_per_w, nchunks, n,
                    h_hbm, src_hbm, dst_hbm, out_hbm,
                    src0_v, src1_v, dst0_v, dst1_v, rows_v, zb_v, acc_s,
                    sg0, sg1, ss, sc0, sc1):
    src_b = (src0_v, src1_v)
    dst_b = (dst0_v, dst1_v)
    scs = (sc0, sc1)
    c = lax.axis_index("c")
    s = lax.axis_index("s")
    wid = c * ns + s
    ebase = pl.multiple_of(wid * e_per_w, e_per_w)
    cedges = _CB * _B
    sgs = (sg0, sg1)

    for i in range(_ZR):
        for k in range(zb_v.shape[1] // 16):
            zb_v[i, pl.ds(k * 16, 16)] = jnp.zeros((16,), jnp.float32)
    nwb = n // _WB
    for k in range((nwb + ns - 1) // ns):
        ci = k * ns + s

        @pl.when(ci < nwb)
        def _():
            r0 = ci * _WB
            for m in range(_WB // _ZR):
                pltpu.sync_copy(zb_v, acc_s.at[pl.ds(r0 + m * _ZR, _ZR)])
    plsc.subcore_barrier()

    def stage(cc, p, sync):
        off = pl.multiple_of(ebase + cc * cedges, cedges)
        if sync:
            pltpu.sync_copy(src_hbm.at[pl.ds(off, cedges)], src_b[p])
            pltpu.sync_copy(dst_hbm.at[wid, cc], dst_b[p])
        else:
            pltpu.async_copy(src_hbm.at[pl.ds(off, cedges)], src_b[p], ss)
            pltpu.async_copy(dst_hbm.at[wid, cc], dst_b[p], ss)

    def drain_stage(p):
        pltpu.make_async_copy(src_hbm.at[pl.ds(ebase, cedges)],
                              src_b[p], ss).wait()
        pltpu.make_async_copy(dst_hbm.at[wid, 0], dst_b[p], ss).wait()

    def process(p):
        def fire(j):
            return pltpu.async_copy(
                h_hbm.at[src_b[p].at[pl.ds(j * _B, _B)]],
                rows_v.at[j % 2], sgs[j % 2])

        gds = [None] * _CB
        sds = [None] * _CB
        gds[0] = fire(0)
        for j in range(_CB):
            if j >= 1:
                sds[j - 1].wait()
            if j + 1 < _CB:
                gds[j + 1] = fire(j + 1)
            gds[j].wait()
            sds[j] = pltpu.async_copy(rows_v.at[j % 2],
                                      acc_s.at[dst_b[p].at[j]],
                                      scs[j % 2], add=True)
        sds[_CB - 1].wait()

    stage(0, 0, sync=True)

    def body(ci2, carry):
        stage(2 * ci2 + 1, 1, sync=False)
        process(0)
        drain_stage(1)

        @pl.when(ci2 < nchunks // 2 - 1)
        def _():
            stage(2 * ci2 + 2, 0, sync=False)
        process(1)

        @pl.when(ci2 < nchunks // 2 - 1)
        def _():
            drain_stage(0)
        return carry

    lax.fori_loop(0, nchunks // 2, body, 0)
    plsc.subcore_barrier()

    for k in range((nwb + ns - 1) // ns):
        ci = k * ns + s

        @pl.when(ci < nwb)
        def _():
            r0 = ci * _WB
            pltpu.sync_copy(acc_s.at[pl.ds(r0, _WB)],
                            out_hbm.at[c, pl.ds(r0, _WB)])


def _sc_segment_sum(h, src, dst4):
    n, d = h.shape
    e = src.shape[0]
    nw, nchunks, cb, b = dst4.shape
    info = plsc.get_sparse_core_info()
    ns = info.num_subcores
    assert nw == _NCU * ns and cb == _CB and b == _B
    assert nchunks % 2 == 0 and n % _WB == 0 and _WB % _ZR == 0
    e_per_w = e // nw
    mesh = plsc.VectorSubcoreMesh(core_axis_name="c", subcore_axis_name="s",
                                  num_cores=_NCU)
    kern = pl.kernel(
        functools.partial(_sc_segsum_body, ns, e_per_w, nchunks, n),
        out_type=jax.ShapeDtypeStruct((_NCU, n, d), jnp.float32),
        mesh=mesh,
        scratch_types=[
            pltpu.VMEM((_CB * _B,), jnp.int32),
            pltpu.VMEM((_CB * _B,), jnp.int32),
            pltpu.VMEM((_CB, _B), jnp.int32),
            pltpu.VMEM((_CB, _B), jnp.int32),
            pltpu.VMEM((2, _B, d), jnp.float32),
            pltpu.VMEM((_ZR, d), jnp.float32),
            pltpu.VMEM_SHARED((n, d), jnp.float32),
            pltpu.SemaphoreType.DMA,
            pltpu.SemaphoreType.DMA,
            pltpu.SemaphoreType.DMA,
            pltpu.SemaphoreType.DMA,
            pltpu.SemaphoreType.DMA,
        ],
    )
    return kern(h, src, dst4)


def kernel(x, edge_index, W_in, b_in, W_rel1, b_rel1, W_root1,
           W_rel2, b_rel2, W_root2):
    e = edge_index.shape[1]
    info = plsc.get_sparse_core_info()
    nw = _NCU * info.num_subcores
    assert e % (nw * _CB * _B) == 0
    src = edge_index[0]
    dst4 = edge_index[1].reshape(nw, e // (nw * _CB * _B), _CB, _B)

    h0 = _tc_linear(x, W_in, b_in)
    p1 = _sc_segment_sum(h0, src, dst4)
    h1 = _tc_combine(p1, h0, W_rel1, b_rel1, W_root1)
    p2 = _sc_segment_sum(h1, src, dst4)
    h2 = _tc_combine(p2, h1, W_rel2, b_rel2, W_root2)
    return h2
